# Initial kernel scaffold; baseline (speedup 1.0000x reference)
import functools

import jax
import jax.numpy as jnp
from jax import lax
from jax.experimental import pallas as pl
from jax.experimental.pallas import tpu as pltpu
from jax.experimental.pallas import tpu_sc as plsc

N = 10000
E = 320000
D = 128

NC = 2
NS = 16
NW = NC * NS
EPT = E // NW
CH = 80
NCHUNK = EPT // CH
RPT = N // NS

_mesh = plsc.VectorSubcoreMesh(core_axis_name="c", subcore_axis_name="s")


@functools.partial(
    pl.kernel,
    out_type=jax.ShapeDtypeStruct((NC, N, D), jnp.float32),
    mesh=_mesh,
    scratch_types=[
        pltpu.VMEM((NCHUNK, CH), jnp.int32),
        pltpu.VMEM((NCHUNK, CH), jnp.int32),
        pltpu.VMEM((CH, D), jnp.float32),
        pltpu.VMEM_SHARED((N, D), jnp.float32),
        pltpu.SemaphoreType.DMA,
    ],
)
def _sc_agg(h_hbm, src_hbm, dst_hbm, zeros_hbm, out_hbm,
            src_v, dst_v, rows_v, acc, sem):
    cid = lax.axis_index("c")
    sid = lax.axis_index("s")
    wid = sid * NC + cid
    pltpu.sync_copy(zeros_hbm.at[pl.ds(sid * RPT, RPT)],
                    acc.at[pl.ds(sid * RPT, RPT)])
    pltpu.sync_copy(src_hbm.at[wid], src_v)
    pltpu.sync_copy(dst_hbm.at[wid], dst_v)
    plsc.subcore_barrier()

    def body(j, carry):
        pltpu.async_copy(h_hbm.at[src_v.at[j]], rows_v, sem).wait()
        pltpu.sync_copy(rows_v, acc.at[dst_v.at[j]], add=True)
        return carry

    lax.fori_loop(0, NCHUNK, body, 0, unroll=False)
    plsc.subcore_barrier()
    pltpu.sync_copy(acc.at[pl.ds(sid * RPT, RPT)],
                    out_hbm.at[cid, pl.ds(sid * RPT, RPT)])


@functools.partial(
    pl.kernel,
    out_type=(jax.ShapeDtypeStruct((NC, N), jnp.float32),
              jax.ShapeDtypeStruct((NC, N), jnp.float32)),
    mesh=_mesh,
    scratch_types=[
        pltpu.VMEM((NCHUNK, CH), jnp.int32),
        pltpu.VMEM((NCHUNK, CH), jnp.int32),
        pltpu.VMEM((CH,), jnp.float32),
        pltpu.VMEM_SHARED((N,), jnp.float32),
        pltpu.VMEM_SHARED((N,), jnp.float32),
    ],
)
def _sc_deg(src_hbm, dst_hbm, zeros1_hbm, deg_out, odeg_out,
            src_v, dst_v, ones_v, dacc, oacc):
    cid = lax.axis_index("c")
    sid = lax.axis_index("s")
    wid = sid * NC + cid
    @pl.when(sid < 10)
    def _():
        pltpu.sync_copy(zeros1_hbm.at[pl.ds(sid * 1000, 1000)],
                        dacc.at[pl.ds(sid * 1000, 1000)])
        pltpu.sync_copy(zeros1_hbm.at[pl.ds(sid * 1000, 1000)],
                        oacc.at[pl.ds(sid * 1000, 1000)])
    pltpu.sync_copy(src_hbm.at[wid], src_v)
    pltpu.sync_copy(dst_hbm.at[wid], dst_v)
    for k in range(CH // 16):
        ones_v[pl.ds(16 * k, 16)] = jnp.ones((16,), jnp.float32)
    plsc.subcore_barrier()

    def body(j, carry):
        pltpu.sync_copy(ones_v, dacc.at[dst_v.at[j]], add=True)
        pltpu.sync_copy(ones_v, oacc.at[src_v.at[j]], add=True)
        return carry

    lax.fori_loop(0, NCHUNK, body, 0, unroll=False)
    plsc.subcore_barrier()
    @pl.when(sid < 10)
    def _():
        pltpu.sync_copy(dacc.at[pl.ds(sid * 1000, 1000)],
                        deg_out.at[cid, pl.ds(sid * 1000, 1000)])
        pltpu.sync_copy(oacc.at[pl.ds(sid * 1000, 1000)],
                        odeg_out.at[cid, pl.ds(sid * 1000, 1000)])


BN = 2000


def _tc_in_proj_body(x_ref, w_ref, b_ref, o_ref):
    o_ref[...] = jnp.dot(x_ref[...], w_ref[...],
                         preferred_element_type=jnp.float32) + b_ref[...]


def _in_proj(x, w, b):
    return pl.pallas_call(
        _tc_in_proj_body,
        grid=(N // BN,),
        in_specs=[
            pl.BlockSpec((BN, D), lambda i: (i, 0)),
            pl.BlockSpec((D, D), lambda i: (0, 0)),
            pl.BlockSpec((1, D), lambda i: (0, 0)),
        ],
        out_specs=pl.BlockSpec((BN, D), lambda i: (i, 0)),
        out_shape=jax.ShapeDtypeStruct((N, D), jnp.float32),
    )(x, w, b.reshape(1, D))


def _tc_layer_body(h_ref, p0_ref, p1_ref, degc_ref,
                   sw_ref, sb_ref, nw_ref, nb_ref,
                   gwh_ref, gwm_ref, gb_ref, g_ref, b_ref, o_ref):
    h = h_ref[...]
    deg = degc_ref[:, 0:1] + degc_ref[:, 1:2]
    has_pred = deg > 0.0
    denom = jnp.maximum(deg, 1.0)
    neigh = (p0_ref[...] + p1_ref[...]) / denom
    hs = jnp.dot(h, sw_ref[...], preferred_element_type=jnp.float32) + sb_ref[...]
    hn = jnp.dot(neigh, nw_ref[...], preferred_element_type=jnp.float32) + nb_ref[...]
    m = hs + jnp.where(has_pred, hn, 0.0)
    gate_lin = (jnp.dot(h, gwh_ref[...], preferred_element_type=jnp.float32)
                + jnp.dot(m, gwm_ref[...], preferred_element_type=jnp.float32)
                + gb_ref[...])
    gate = jax.nn.sigmoid(gate_lin)
    v = gate * m + (1.0 - gate) * h
    mu = jnp.mean(v, axis=-1, keepdims=True)
    var = jnp.mean((v - mu) ** 2, axis=-1, keepdims=True)
    y = (v - mu) * lax.rsqrt(var + 1e-5) * g_ref[...] + b_ref[...]
    o_ref[...] = jnp.maximum(y, 0.0)


def _tc_layer(h, p0, p1, deg_cols, lp):
    row = lambda i: (i, 0)
    full = lambda i: (0, 0)
    return pl.pallas_call(
        _tc_layer_body,
        grid=(N // BN,),
        in_specs=[
            pl.BlockSpec((BN, D), row),
            pl.BlockSpec((BN, D), row),
            pl.BlockSpec((BN, D), row),
            pl.BlockSpec((BN, NC), row),
            pl.BlockSpec((D, D), full),
            pl.BlockSpec((1, D), full),
            pl.BlockSpec((D, D), full),
            pl.BlockSpec((1, D), full),
            pl.BlockSpec((D, D), full),
            pl.BlockSpec((D, D), full),
            pl.BlockSpec((1, D), full),
            pl.BlockSpec((1, D), full),
            pl.BlockSpec((1, D), full),
        ],
        out_specs=pl.BlockSpec((BN, D), row),
        out_shape=jax.ShapeDtypeStruct((N, D), jnp.float32),
    )(h, p0, p1, deg_cols,
      lp['self_w'], lp['self_b'].reshape(1, D),
      lp['neigh_w'], lp['neigh_b'].reshape(1, D),
      lp['gate_w'][:D], lp['gate_w'][D:], lp['gate_b'].reshape(1, D),
      lp['ln_g'].reshape(1, D), lp['ln_b'].reshape(1, D))


def _tc_pool_body(h_ref, odegc_ref, aw_ref, ab_ref, scw_ref, scb_ref, o_ref):
    h = h_ref[...]
    a = jnp.tanh(jnp.dot(h, aw_ref[...], preferred_element_type=jnp.float32)
                 + ab_ref[...])
    s = jnp.dot(a, scw_ref[...], preferred_element_type=jnp.float32) + scb_ref[...]
    od = odegc_ref[:, 0:1] + odegc_ref[:, 1:2]
    is_sink = od == 0.0
    any_sink = jnp.any(is_sink)
    mask = is_sink | jnp.logical_not(any_sink)
    s = jnp.where(mask, s, -1e30)
    mx = jnp.max(s)
    ex = jnp.exp(s - mx)
    w = ex / jnp.sum(ex)
    o_ref[...] = jnp.sum(w * h, axis=0, keepdims=True)


def _pool(h, odeg_cols, aw, ab, scw, scb):
    full = lambda: (0, 0)
    return pl.pallas_call(
        _tc_pool_body,
        grid=(),
        in_specs=[
            pl.BlockSpec((N, D), full),
            pl.BlockSpec((N, NC), full),
            pl.BlockSpec((D, D), full),
            pl.BlockSpec((1, D), full),
            pl.BlockSpec((D, 1), full),
            pl.BlockSpec((1, 1), full),
        ],
        out_specs=pl.BlockSpec((1, D), full),
        out_shape=jax.ShapeDtypeStruct((1, D), jnp.float32),
    )(h, odeg_cols, aw, ab.reshape(1, D), scw, scb.reshape(1, 1))


def kernel(node_feats, params, edge_index):
    src = edge_index[0].reshape(NW, NCHUNK, CH)
    dst = edge_index[1].reshape(NW, NCHUNK, CH)
    zeros2 = jnp.zeros((N, D), jnp.float32)
    zeros1 = jnp.zeros((N,), jnp.float32)

    deg_p, odeg_p = _sc_deg(src, dst, zeros1)
    deg_cols = jnp.transpose(deg_p, (1, 0))
    odeg_cols = jnp.transpose(odeg_p, (1, 0))

    h = _in_proj(node_feats, params['in_w'], params['in_b'])
    for lp in params['layers']:
        parts = _sc_agg(h, src, dst, zeros2)
        h = _tc_layer(h, parts[0], parts[1], deg_cols, lp)

    emb = _pool(h, odeg_cols, params['att_w'], params['att_b'],
                params['score_w'], params['score_b'])
    return h, emb.reshape(D)


# speedup vs baseline: 6.3937x; 6.3937x over previous
import functools

import jax
import jax.numpy as jnp
from jax import lax
from jax.experimental import pallas as pl
from jax.experimental.pallas import tpu as pltpu
from jax.experimental.pallas import tpu_sc as plsc

N = 10000
E = 320000
D = 128

NC = 2
NS = 16
NW = NC * NS
EPT = E // NW
CH = 80
NCHUNK = EPT // CH
NP = 10240
RPT = NP // NS

_mesh = plsc.VectorSubcoreMesh(core_axis_name="c", subcore_axis_name="s")


@functools.partial(
    pl.kernel,
    out_type=jax.ShapeDtypeStruct((NC, NP, D), jnp.float32),
    mesh=_mesh,
    scratch_types=[
        pltpu.VMEM((NCHUNK, CH), jnp.int32),
        pltpu.VMEM((NCHUNK, CH), jnp.int32),
        pltpu.VMEM((CH, D), jnp.float32),
        pltpu.VMEM_SHARED((NP, D), jnp.float32),
        pltpu.SemaphoreType.DMA,
    ],
)
def _sc_agg(h_hbm, src_hbm, dst_hbm, zeros_hbm, out_hbm,
            src_v, dst_v, rows_v, acc, sem):
    cid = lax.axis_index("c")
    sid = lax.axis_index("s")
    wid = sid * NC + cid
    pltpu.sync_copy(zeros_hbm, acc.at[pl.ds(sid * RPT, RPT)])
    pltpu.sync_copy(src_hbm.at[wid], src_v)
    pltpu.sync_copy(dst_hbm.at[wid], dst_v)
    plsc.subcore_barrier()

    def body(j, carry):
        pltpu.async_copy(h_hbm.at[src_v.at[j]], rows_v, sem).wait()
        pltpu.sync_copy(rows_v, acc.at[dst_v.at[j]], add=True)
        return carry

    lax.fori_loop(0, NCHUNK, body, 0, unroll=False)
    plsc.subcore_barrier()
    pltpu.sync_copy(acc.at[pl.ds(sid * RPT, RPT)],
                    out_hbm.at[cid, pl.ds(sid * RPT, RPT)])


NP = 10240


@functools.partial(
    pl.kernel,
    out_type=(jax.ShapeDtypeStruct((NC * NP,), jnp.float32),
              jax.ShapeDtypeStruct((NC * NP,), jnp.float32)),
    mesh=_mesh,
    scratch_types=[
        pltpu.VMEM((NCHUNK, CH), jnp.int32),
        pltpu.VMEM((NCHUNK, CH), jnp.int32),
        pltpu.VMEM((CH,), jnp.float32),
        pltpu.VMEM((NP // NS,), jnp.float32),
        pltpu.VMEM_SHARED((NP,), jnp.float32),
        pltpu.VMEM_SHARED((NP,), jnp.float32),
    ],
)
def _sc_deg(src_hbm, dst_hbm, deg_out, odeg_out,
            src_v, dst_v, ones_v, zbuf, dacc, oacc):
    cid = lax.axis_index("c")
    sid = lax.axis_index("s")
    wid = sid * NC + cid
    seg = NP // NS
    for k in range(seg // 16):
        zbuf[pl.ds(16 * k, 16)] = jnp.zeros((16,), jnp.float32)
    pltpu.sync_copy(zbuf, dacc.at[pl.ds(sid * seg, seg)])
    pltpu.sync_copy(zbuf, oacc.at[pl.ds(sid * seg, seg)])
    pltpu.sync_copy(src_hbm.at[wid], src_v)
    pltpu.sync_copy(dst_hbm.at[wid], dst_v)
    for k in range(CH // 16):
        ones_v[pl.ds(16 * k, 16)] = jnp.ones((16,), jnp.float32)
    plsc.subcore_barrier()

    def body(j, carry):
        pltpu.sync_copy(ones_v, dacc.at[dst_v.at[j]], add=True)
        pltpu.sync_copy(ones_v, oacc.at[src_v.at[j]], add=True)
        return carry

    lax.fori_loop(0, NCHUNK, body, 0, unroll=False)
    plsc.subcore_barrier()
    pltpu.sync_copy(dacc.at[pl.ds(sid * seg, seg)], zbuf)
    pltpu.sync_copy(zbuf, deg_out.at[pl.ds(cid * NP + sid * seg, seg)])
    pltpu.sync_copy(oacc.at[pl.ds(sid * seg, seg)], zbuf)
    pltpu.sync_copy(zbuf, odeg_out.at[pl.ds(cid * NP + sid * seg, seg)])


BN = 2000


def _tc_in_proj_body(x_ref, w_ref, b_ref, o_ref):
    o_ref[...] = jnp.dot(x_ref[...], w_ref[...],
                         preferred_element_type=jnp.float32) + b_ref[...]


def _in_proj(x, w, b):
    return pl.pallas_call(
        _tc_in_proj_body,
        grid=(N // BN,),
        in_specs=[
            pl.BlockSpec((BN, D), lambda i: (i, 0)),
            pl.BlockSpec((D, D), lambda i: (0, 0)),
            pl.BlockSpec((1, D), lambda i: (0, 0)),
        ],
        out_specs=pl.BlockSpec((BN, D), lambda i: (i, 0)),
        out_shape=jax.ShapeDtypeStruct((N, D), jnp.float32),
    )(x, w, b.reshape(1, D))


def _tc_layer_body(h_ref, p0_ref, p1_ref, degc_ref,
                   sw_ref, sb_ref, nw_ref, nb_ref,
                   gwh_ref, gwm_ref, gb_ref, g_ref, b_ref, o_ref):
    h = h_ref[...]
    deg = degc_ref[:, 0:1] + degc_ref[:, 1:2]
    has_pred = deg > 0.0
    denom = jnp.maximum(deg, 1.0)
    neigh = (p0_ref[...] + p1_ref[...]) / denom
    hs = jnp.dot(h, sw_ref[...], preferred_element_type=jnp.float32) + sb_ref[...]
    hn = jnp.dot(neigh, nw_ref[...], preferred_element_type=jnp.float32) + nb_ref[...]
    m = hs + jnp.where(has_pred, hn, 0.0)
    gate_lin = (jnp.dot(h, gwh_ref[...], preferred_element_type=jnp.float32)
                + jnp.dot(m, gwm_ref[...], preferred_element_type=jnp.float32)
                + gb_ref[...])
    gate = jax.nn.sigmoid(gate_lin)
    v = gate * m + (1.0 - gate) * h
    mu = jnp.mean(v, axis=-1, keepdims=True)
    var = jnp.mean((v - mu) ** 2, axis=-1, keepdims=True)
    y = (v - mu) * lax.rsqrt(var + 1e-5) * g_ref[...] + b_ref[...]
    o_ref[...] = jnp.maximum(y, 0.0)


def _tc_layer(h, p0, p1, deg_cols, lp):
    row = lambda i: (i, 0)
    full = lambda i: (0, 0)
    return pl.pallas_call(
        _tc_layer_body,
        grid=(N // BN,),
        in_specs=[
            pl.BlockSpec((BN, D), row),
            pl.BlockSpec((BN, D), row),
            pl.BlockSpec((BN, D), row),
            pl.BlockSpec((BN, NC), row),
            pl.BlockSpec((D, D), full),
            pl.BlockSpec((1, D), full),
            pl.BlockSpec((D, D), full),
            pl.BlockSpec((1, D), full),
            pl.BlockSpec((D, D), full),
            pl.BlockSpec((D, D), full),
            pl.BlockSpec((1, D), full),
            pl.BlockSpec((1, D), full),
            pl.BlockSpec((1, D), full),
        ],
        out_specs=pl.BlockSpec((BN, D), row),
        out_shape=jax.ShapeDtypeStruct((N, D), jnp.float32),
    )(h, p0, p1, deg_cols,
      lp['self_w'], lp['self_b'].reshape(1, D),
      lp['neigh_w'], lp['neigh_b'].reshape(1, D),
      lp['gate_w'][:D], lp['gate_w'][D:], lp['gate_b'].reshape(1, D),
      lp['ln_g'].reshape(1, D), lp['ln_b'].reshape(1, D))


def _tc_pool_body(h_ref, odegc_ref, aw_ref, ab_ref, scw_ref, scb_ref, o_ref):
    h = h_ref[...]
    a = jnp.tanh(jnp.dot(h, aw_ref[...], preferred_element_type=jnp.float32)
                 + ab_ref[...])
    s = jnp.dot(a, scw_ref[...], preferred_element_type=jnp.float32) + scb_ref[...]
    od = odegc_ref[:, 0:1] + odegc_ref[:, 1:2]
    is_sink = od == 0.0
    any_sink = jnp.any(is_sink)
    mask = is_sink | jnp.logical_not(any_sink)
    s = jnp.where(mask, s, -1e30)
    mx = jnp.max(s)
    ex = jnp.exp(s - mx)
    w = ex / jnp.sum(ex)
    o_ref[...] = jnp.sum(w * h, axis=0, keepdims=True)


def _pool(h, odeg_cols, aw, ab, scw, scb):
    full = lambda: (0, 0)
    return pl.pallas_call(
        _tc_pool_body,
        grid=(),
        in_specs=[
            pl.BlockSpec((N, D), full),
            pl.BlockSpec((N, NC), full),
            pl.BlockSpec((D, D), full),
            pl.BlockSpec((1, D), full),
            pl.BlockSpec((D, 1), full),
            pl.BlockSpec((1, 1), full),
        ],
        out_specs=pl.BlockSpec((1, D), full),
        out_shape=jax.ShapeDtypeStruct((1, D), jnp.float32),
    )(h, odeg_cols, aw, ab.reshape(1, D), scw, scb.reshape(1, 1))


def kernel(node_feats, params, edge_index):
    src = edge_index[0].reshape(NW, NCHUNK, CH)
    dst = edge_index[1].reshape(NW, NCHUNK, CH)
    zeros2 = jnp.zeros((RPT, D), jnp.float32)

    deg_p, odeg_p = _sc_deg(src, dst)
    deg_cols = jnp.transpose(deg_p.reshape(NC, NP)[:, :N], (1, 0))
    odeg_cols = jnp.transpose(odeg_p.reshape(NC, NP)[:, :N], (1, 0))

    h = _in_proj(node_feats, params['in_w'], params['in_b'])
    for lp in params['layers']:
        parts = _sc_agg(h, src, dst, zeros2)
        h = _tc_layer(h, parts[0, :N], parts[1, :N], deg_cols, lp)

    emb = _pool(h, odeg_cols, params['att_w'], params['att_b'],
                params['score_w'], params['score_b'])
    return h, emb.reshape(D)
